# baseline (speedup 1.0000x reference)
import functools

import jax
import jax.numpy as jnp
from jax import lax
from jax.experimental import pallas as pl
from jax.experimental.pallas import tpu as pltpu
from jax.experimental.pallas import tpu_sc as plsc

_CHUNK = 40
_NJOB = 5
_D = 128
_K = 3 * _D

_mesh = plsc.VectorSubcoreMesh(core_axis_name="c", subcore_axis_name="s",
                               num_cores=1)


_TPM = 5


@functools.partial(
    pl.kernel,
    mesh=_mesh,
    out_type=jax.ShapeDtypeStruct((_NJOB, _K), jnp.float32),
    scratch_types=[
        pltpu.VMEM((_CHUNK,), jnp.int32),
        pltpu.VMEM((_CHUNK,), jnp.int32),
        pltpu.VMEM((_CHUNK,), jnp.int32),
        pltpu.VMEM((_CHUNK, _D), jnp.float32),
        pltpu.VMEM((_CHUNK, _D), jnp.float32),
        pltpu.VMEM((_CHUNK, _D), jnp.float32),
        pltpu.VMEM((_K,), jnp.float32),
        pltpu.SemaphoreType.DMA,
    ],
)
def _gather_pool(pat_hbm, e0, e1, e2, out_hbm,
                 idx0_v, idx1_v, idx2_v, rows0_v, rows1_v, rows2_v,
                 acc_v, sem_g):
    s = lax.axis_index("s")

    @pl.when(s < _NJOB)
    def _():
        off = pl.multiple_of(s * _CHUNK, 8)
        pltpu.sync_copy(pat_hbm.at[pl.ds(600 + off, _CHUNK)], idx0_v)
        pltpu.sync_copy(pat_hbm.at[pl.ds(800 + off, _CHUNK)], idx1_v)
        pltpu.sync_copy(pat_hbm.at[pl.ds(400 + off, _CHUNK)], idx2_v)
        g0 = pltpu.async_copy(e0.at[idx0_v], rows0_v, sem_g)
        g1 = pltpu.async_copy(e1.at[idx1_v], rows1_v, sem_g)
        g2 = pltpu.async_copy(e2.at[idx2_v], rows2_v, sem_g)
        g0.wait()
        g1.wait()
        g2.wait()
        for m, rv in enumerate((rows0_v, rows1_v, rows2_v)):
            for v in range(_D // 16):
                a = rv[0, pl.ds(v * 16, 16)]
                for r in range(1, _CHUNK):
                    a = a + rv[r, pl.ds(v * 16, 16)]
                acc_v[pl.ds(m * _D + v * 16, 16)] = a
        pltpu.sync_copy(acc_v, out_hbm.at[s])


def _dense(partial_ref, w_ref, b_ref, ddi_ref, res_ref, bn_ref):
    rep = jnp.sum(partial_ref[:], axis=0, keepdims=True)
    rep = jnp.maximum(rep, 0.0)
    out = lax.dot_general(
        rep, w_ref[:],
        dimension_numbers=(((1,), (1,)), ((), ())),
        preferred_element_type=jnp.float32,
    ) + b_ref[:]
    res_ref[:] = out
    neg = jax.nn.sigmoid(out)
    s = jnp.sum(neg)
    bn_ref[:] = jnp.reshape(0.0005 * ddi_ref[0, 0] * s * s, (1, 1))


def kernel(patient, E0, E1, E2, W, b, ddi_adj):
    partial = _gather_pool(patient.reshape(-1), E0, E1, E2)
    result, bn = pl.pallas_call(
        _dense,
        out_shape=(
            jax.ShapeDtypeStruct((1, 1000), jnp.float32),
            jax.ShapeDtypeStruct((1, 1), jnp.float32),
        ),
    )(partial, W, b.reshape(1, 1000), ddi_adj)
    return (result, bn.reshape(()))
